# Initial kernel scaffold; baseline (speedup 1.0000x reference)
import jax
import jax.numpy as jnp
from jax.experimental import pallas as pl


def kernel(x, table0, table1, proj_w, proj_b):
    raise NotImplementedError("write your pallas kernel here")


# speedup vs baseline: 4.5353x; 4.5353x over previous
import functools

import jax
import jax.numpy as jnp
import numpy as np
from jax import lax
from jax.experimental import pallas as pl
from jax.experimental.pallas import tpu as pltpu
from jax.experimental.pallas import tpu_sc as plsc

NUM_FIELDS = 26
N0 = 13
BATCH = 4096
D0 = 64
D1 = 16
NC, NS, L = 2, 16, 16
NW = NC * NS
RPW = BATCH // NW
ROWS_PER_SC = NS * RPW

_FIELD_DIMS = np.full(NUM_FIELDS, 100000, dtype=np.int64)
_OFF0 = np.concatenate([[0], np.cumsum(_FIELD_DIMS[:N0])[:-1]]).astype(np.int64)
_OFF1 = np.concatenate([[0], np.cumsum(_FIELD_DIMS[N0:])[:-1]]).astype(np.int64)
_OFFS = [int(v) for v in np.concatenate([_OFF0, _OFF1])]

_mesh = plsc.VectorSubcoreMesh(core_axis_name="c", subcore_axis_name="s")


@functools.partial(
    pl.kernel,
    mesh=_mesh,
    out_type=(
        jax.ShapeDtypeStruct((BATCH, D0), jnp.float32),
        jax.ShapeDtypeStruct((BATCH, D1), jnp.float32),
    ),
    scratch_types=[
        pltpu.VMEM((NUM_FIELDS, RPW), jnp.int32),
        pltpu.VMEM((RPW,), jnp.int32),
        pltpu.VMEM((2, RPW, D0), jnp.float32),
        pltpu.VMEM((2, RPW, D1), jnp.float32),
        pltpu.VMEM_SHARED((ROWS_PER_SC, D0), jnp.float32),
        pltpu.VMEM_SHARED((ROWS_PER_SC, D1), jnp.float32),
        pltpu.SemaphoreType.DMA((2,)),
    ],
    compiler_params=pltpu.CompilerParams(use_tc_tiling_on_sc=False),
)
def _pool_sc(xt, table0, table1, out0, out1,
             idx_v, rid_v, buf0, buf1, acc0, acc1, gsem):
    cid = lax.axis_index("c")
    sid = lax.axis_index("s")
    wid = cid * NS + sid
    base = wid * RPW
    sbase = sid * RPW

    pltpu.sync_copy(xt.at[:, pl.ds(base, RPW)], idx_v)

    for g in range(RPW // L):
        sl = pl.ds(g * L, L)
        rid_v[sl] = lax.iota(jnp.int32, L) + (sbase + g * L)
        for f in range(NUM_FIELDS):
            if _OFFS[f]:
                idx_v[f, sl] = idx_v[f, sl] + _OFFS[f]

    def start_gather(f):
        slot = f % 2
        if f < N0:
            return pltpu.async_copy(table0.at[idx_v.at[f]], buf0.at[slot],
                                    gsem.at[slot])
        return pltpu.async_copy(table1.at[idx_v.at[f]], buf1.at[slot],
                                gsem.at[slot])

    cpy = start_gather(0)
    for f in range(NUM_FIELDS):
        ncpy = start_gather(f + 1) if f + 1 < NUM_FIELDS else None
        cpy.wait()
        slot = f % 2
        if f < N0:
            if f == 0:
                pltpu.sync_copy(buf0.at[slot], acc0.at[pl.ds(sbase, RPW)])
            else:
                pltpu.sync_copy(buf0.at[slot], acc0.at[rid_v], add=True)
        else:
            if f == N0:
                pltpu.sync_copy(buf1.at[slot], acc1.at[pl.ds(sbase, RPW)])
            else:
                pltpu.sync_copy(buf1.at[slot], acc1.at[rid_v], add=True)
        cpy = ncpy

    pltpu.sync_copy(acc0.at[pl.ds(sbase, RPW)], out0.at[pl.ds(base, RPW)])
    pltpu.sync_copy(acc1.at[pl.ds(sbase, RPW)], out1.at[pl.ds(base, RPW)])


def _proj_body(p0_ref, p1_ref, w_ref, b_ref, o_ref):
    proj = lax.dot_general(p1_ref[...], w_ref[...],
                           (((1,), (1,)), ((), ())),
                           preferred_element_type=jnp.float32)
    o_ref[...] = p0_ref[...] + proj + float(N0) * b_ref[...]


_PROJ_GRID = 8
_PB = BATCH // _PROJ_GRID


def _project_tc(pooled0, pooled1, proj_w, proj_b2d):
    return pl.pallas_call(
        _proj_body,
        grid=(_PROJ_GRID,),
        in_specs=[
            pl.BlockSpec((_PB, D0), lambda i: (i, 0)),
            pl.BlockSpec((_PB, D1), lambda i: (i, 0)),
            pl.BlockSpec((D0, D1), lambda i: (0, 0)),
            pl.BlockSpec((1, D0), lambda i: (0, 0)),
        ],
        out_specs=pl.BlockSpec((_PB, D0), lambda i: (i, 0)),
        out_shape=jax.ShapeDtypeStruct((BATCH, D0), jnp.float32),
    )(pooled0, pooled1, proj_w, proj_b2d)


def kernel(x, table0, table1, proj_w, proj_b):
    xt = x.T.astype(jnp.int32)
    pooled0, pooled1 = _pool_sc(xt, table0, table1)
    return _project_tc(pooled0, pooled1, proj_w, proj_b.reshape(1, D0))
